# Initial kernel scaffold; baseline (speedup 1.0000x reference)
import jax
import jax.numpy as jnp
from jax.experimental import pallas as pl


def kernel(x, w_k, w_v, w_q, w_o, ln_k_g, ln_k_b, ln_v_g, ln_v_b, ln_q_g, ln_q_b, fc1_w, fc1_b, fc2_w, fc2_b, lr_scale, momentum_scale, gate_w, gate_b):
    raise NotImplementedError("write your pallas kernel here")


# speedup vs baseline: 2.8501x; 2.8501x over previous
import functools
import math

import jax
import jax.numpy as jnp
from jax.experimental import pallas as pl
from jax.experimental.pallas import tpu as pltpu

DIM = 512
HEAD_DIM = 64
NUM_HEADS = 4
HID = 4 * HEAD_DIM
BASE_LR = 0.1
BASE_MOM = 0.9
LN_EPS = 1e-5
SQ2PI = math.sqrt(2.0 / math.pi)
INVSQ2 = 1.0 / math.sqrt(2.0)

_HIGHEST = jax.lax.Precision.HIGHEST


def _scan_kernel(x_ref, wk_ref, wv_ref, wq_ref, lnkg, lnkb, lnvg, lnvb,
                 lnqg, lnqb, w1t0_ref, b10_ref, w20_ref, b20_ref, lr_ref,
                 mom_ref, y_ref, k_scr, v_scr, q_scr, w1t_s, w2_s, mw1t_s,
                 mw2_s, b1_s, b2_s, mb1_s, mb2_s, *, batch, seq):
    d = HEAD_DIM
    hid = HID

    xb = x_ref[...]

    def proj(w_ref, g_ref, bb_ref, scr):
        t = jax.lax.dot_general(xb, w_ref[...], (((1,), (1,)), ((), ())),
                                precision=_HIGHEST,
                                preferred_element_type=jnp.float32)
        mu = jnp.mean(t, axis=1, keepdims=True)
        var = jnp.mean((t - mu) ** 2, axis=1, keepdims=True)
        tn = (t - mu) / jnp.sqrt(var + LN_EPS) * g_ref[...] + bb_ref[...]
        scr[...] = tn.reshape(seq, batch, d)

    proj(wk_ref, lnkg, lnkb, k_scr)
    proj(wv_ref, lnvg, lnvb, v_scr)
    proj(wq_ref, lnqg, lnqb, q_scr)

    lr = BASE_LR * jax.nn.sigmoid(lr_ref[0, 0, 0])
    mom = BASE_MOM * jax.nn.sigmoid(mom_ref[0, 0, 0])

    w1t_s[...] = jnp.broadcast_to(w1t0_ref[...], (batch, d, hid))
    w2_s[...] = jnp.broadcast_to(w20_ref[...], (batch, d, hid))
    b1_s[...] = jnp.broadcast_to(b10_ref[0], (batch, hid))
    b2_s[...] = jnp.broadcast_to(b20_ref[0], (batch, d))
    mw1t_s[...] = jnp.zeros((batch, d, hid), jnp.float32)
    mw2_s[...] = jnp.zeros((batch, d, hid), jnp.float32)
    mb1_s[...] = jnp.zeros((batch, hid), jnp.float32)
    mb2_s[...] = jnp.zeros((batch, d), jnp.float32)

    def step(it, carry):
        kt = k_scr[it]
        vt = v_scr[it]
        qt = q_scr[it]
        w1t = w1t_s[...]
        w2 = w2_s[...]
        b1 = b1_s[...]
        b2 = b2_s[...]

        h1 = jnp.sum(kt[:, :, None] * w1t, axis=1) + b1
        cdf = 0.5 * (1.0 + jax.lax.erf(h1 * INVSQ2))
        a = h1 * cdf
        out = jnp.sum(a[:, None, :] * w2, axis=2) + b2
        d_out = out - vt
        d_a = jnp.sum(d_out[:, :, None] * w2, axis=1)
        pdf = SQ2PI * jnp.exp(-0.5 * h1 * h1)
        d_h = d_a * (cdf + h1 * pdf)

        mw1t = mom * mw1t_s[...] + kt[:, :, None] * d_h[:, None, :]
        mw2 = mom * mw2_s[...] + d_out[:, :, None] * a[:, None, :]
        mb1 = mom * mb1_s[...] + d_h
        mb2 = mom * mb2_s[...] + d_out
        w1t_n = w1t - lr * mw1t
        w2_n = w2 - lr * mw2
        b1_n = b1 - lr * mb1
        b2_n = b2 - lr * mb2
        mw1t_s[...] = mw1t
        mw2_s[...] = mw2
        mb1_s[...] = mb1
        mb2_s[...] = mb2
        w1t_s[...] = w1t_n
        w2_s[...] = w2_n
        b1_s[...] = b1_n
        b2_s[...] = b2_n

        h1q = jnp.sum(qt[:, :, None] * w1t_n, axis=1) + b1_n
        cdfq = 0.5 * (1.0 + jax.lax.erf(h1q * INVSQ2))
        aq = h1q * cdfq
        yq = jnp.sum(aq[:, None, :] * w2_n, axis=2) + b2_n
        y_ref[0, pl.ds(it, 1)] = yq[None]
        return carry

    jax.lax.fori_loop(0, seq, step, 0)


def _out_kernel(r_ref, x_ref, wo_ref, gw_ref, gb_ref, o_ref):
    rp = jax.lax.dot_general(r_ref[...], wo_ref[...], (((1,), (1,)), ((), ())),
                             precision=_HIGHEST,
                             preferred_element_type=jnp.float32)
    gt = jax.lax.dot_general(x_ref[...], gw_ref[...], (((1,), (1,)), ((), ())),
                             precision=_HIGHEST,
                             preferred_element_type=jnp.float32) + gb_ref[...]
    o_ref[...] = rp * jax.nn.sigmoid(gt)


def kernel(x, w_k, w_v, w_q, w_o, ln_k_g, ln_k_b, ln_v_g, ln_v_b, ln_q_g,
           ln_q_b, fc1_w, fc1_b, fc2_w, fc2_b, lr_scale, momentum_scale,
           gate_w, gate_b):
    batch, seq, dim = x.shape
    h = NUM_HEADS
    d = HEAD_DIM
    hid = HID

    x_tb = jnp.transpose(x, (1, 0, 2)).reshape(seq * batch, dim)
    fc1_wt = jnp.swapaxes(fc1_w, 1, 2)
    fc1_b3 = fc1_b.reshape(h, 1, hid)
    fc2_b3 = fc2_b.reshape(h, 1, d)
    lr2 = lr_scale.reshape(h, 1, 1)
    mom2 = momentum_scale.reshape(h, 1, 1)
    g2 = lambda v: v.reshape(1, d)

    scan = pl.pallas_call(
        functools.partial(_scan_kernel, batch=batch, seq=seq),
        grid=(h,),
        in_specs=[
            pl.BlockSpec((seq * batch, dim), lambda i: (0, 0)),
            pl.BlockSpec((d, dim), lambda i: (i, 0)),
            pl.BlockSpec((d, dim), lambda i: (i, 0)),
            pl.BlockSpec((d, dim), lambda i: (i, 0)),
            pl.BlockSpec((1, d), lambda i: (0, 0)),
            pl.BlockSpec((1, d), lambda i: (0, 0)),
            pl.BlockSpec((1, d), lambda i: (0, 0)),
            pl.BlockSpec((1, d), lambda i: (0, 0)),
            pl.BlockSpec((1, d), lambda i: (0, 0)),
            pl.BlockSpec((1, d), lambda i: (0, 0)),
            pl.BlockSpec((1, d, hid), lambda i: (i, 0, 0)),
            pl.BlockSpec((1, 1, hid), lambda i: (i, 0, 0)),
            pl.BlockSpec((1, d, hid), lambda i: (i, 0, 0)),
            pl.BlockSpec((1, 1, d), lambda i: (i, 0, 0)),
            pl.BlockSpec((1, 1, 1), lambda i: (i, 0, 0)),
            pl.BlockSpec((1, 1, 1), lambda i: (i, 0, 0)),
        ],
        out_specs=pl.BlockSpec((1, seq, batch, d), lambda i: (i, 0, 0, 0)),
        out_shape=jax.ShapeDtypeStruct((h, seq, batch, d), jnp.float32),
        scratch_shapes=[
            pltpu.VMEM((seq, batch, d), jnp.float32),
            pltpu.VMEM((seq, batch, d), jnp.float32),
            pltpu.VMEM((seq, batch, d), jnp.float32),
            pltpu.VMEM((batch, d, hid), jnp.float32),
            pltpu.VMEM((batch, d, hid), jnp.float32),
            pltpu.VMEM((batch, d, hid), jnp.float32),
            pltpu.VMEM((batch, d, hid), jnp.float32),
            pltpu.VMEM((batch, hid), jnp.float32),
            pltpu.VMEM((batch, d), jnp.float32),
            pltpu.VMEM((batch, hid), jnp.float32),
            pltpu.VMEM((batch, d), jnp.float32),
        ],
        compiler_params=pltpu.CompilerParams(
            dimension_semantics=("parallel",),
            vmem_limit_bytes=100 * 1024 * 1024,
        ),
    )
    ys = scan(x_tb, w_k, w_v, w_q, g2(ln_k_g), g2(ln_k_b), g2(ln_v_g),
              g2(ln_v_b), g2(ln_q_g), g2(ln_q_b), fc1_wt, fc1_b3, fc2_w,
              fc2_b3, lr2, mom2)

    r = jnp.transpose(ys, (2, 1, 0, 3)).reshape(batch * seq, h * d)
    x2 = x.reshape(batch * seq, dim)

    blk = 256
    out = pl.pallas_call(
        _out_kernel,
        grid=(batch * seq // blk,),
        in_specs=[
            pl.BlockSpec((blk, h * d), lambda i: (i, 0)),
            pl.BlockSpec((blk, dim), lambda i: (i, 0)),
            pl.BlockSpec((dim, h * d), lambda i: (0, 0)),
            pl.BlockSpec((dim, dim), lambda i: (0, 0)),
            pl.BlockSpec((1, dim), lambda i: (0, 0)),
        ],
        out_specs=pl.BlockSpec((blk, dim), lambda i: (i, 0)),
        out_shape=jax.ShapeDtypeStruct((batch * seq, dim), jnp.float32),
        compiler_params=pltpu.CompilerParams(
            dimension_semantics=("parallel",),
        ),
    )(r, x2, w_o, gate_w, gate_b.reshape(1, dim))
    return out.reshape(batch, seq, dim)


# speedup vs baseline: 3.4115x; 1.1970x over previous
import functools
import math

import jax
import jax.numpy as jnp
from jax.experimental import pallas as pl
from jax.experimental.pallas import tpu as pltpu

DIM = 512
HEAD_DIM = 64
NUM_HEADS = 4
HID = 4 * HEAD_DIM
BASE_LR = 0.1
BASE_MOM = 0.9
LN_EPS = 1e-5
SQ2PI = math.sqrt(2.0 / math.pi)
INVSQ2 = 1.0 / math.sqrt(2.0)

_HIGHEST = jax.lax.Precision.HIGHEST


def _scan_kernel(x_ref, wk_ref, wv_ref, wq_ref, lnkg, lnkb, lnvg, lnvb,
                 lnqg, lnqb, w1t0_ref, b10_ref, w20_ref, b20_ref, lr_ref,
                 mom_ref, y_ref, k_scr, v_scr, q_scr, w1v_s, w2v_s, mw1v_s,
                 mw2v_s, b1_s, b2bd_s, mb1_s, mb2bd_s, *, batch, seq):
    d = HEAD_DIM
    hid = HID
    p = 2 * batch
    kdim = p * d

    xb = x_ref[...]

    def proj(w_ref, g_ref, bb_ref, scr):
        for h2 in range(2):
            w = w_ref[h2 * d:(h2 + 1) * d, :]
            t = jax.lax.dot_general(xb, w, (((1,), (1,)), ((), ())),
                                    precision=_HIGHEST,
                                    preferred_element_type=jnp.float32)
            mu = jnp.mean(t, axis=1, keepdims=True)
            var = jnp.mean((t - mu) ** 2, axis=1, keepdims=True)
            tn = (t - mu) / jnp.sqrt(var + LN_EPS) * g_ref[...] + bb_ref[...]
            scr[:, h2 * batch:(h2 + 1) * batch, :] = tn.reshape(seq, batch, d)

    proj(wk_ref, lnkg, lnkb, k_scr)
    proj(wv_ref, lnvg, lnvb, v_scr)
    proj(wq_ref, lnqg, lnqb, q_scr)

    lr_a = BASE_LR * jax.nn.sigmoid(lr_ref[0, 0, 0])
    lr_b = BASE_LR * jax.nn.sigmoid(lr_ref[1, 0, 0])
    mom_a = BASE_MOM * jax.nn.sigmoid(mom_ref[0, 0, 0])
    mom_b = BASE_MOM * jax.nn.sigmoid(mom_ref[1, 0, 0])

    for i in range(batch):
        w1v_s[i * d:(i + 1) * d] = w1t0_ref[0]
        w1v_s[(batch + i) * d:(batch + i + 1) * d] = w1t0_ref[1]
        w2v_s[i * d:(i + 1) * d] = w20_ref[0]
        w2v_s[(batch + i) * d:(batch + i + 1) * d] = w20_ref[1]
    mw1v_s[...] = jnp.zeros((kdim, hid), jnp.float32)
    mw2v_s[...] = jnp.zeros((kdim, hid), jnp.float32)
    b1_s[0:batch] = jnp.broadcast_to(b10_ref[0], (batch, hid))
    b1_s[batch:p] = jnp.broadcast_to(b10_ref[1], (batch, hid))
    mb1_s[...] = jnp.zeros((p, hid), jnp.float32)
    mb2bd_s[...] = jnp.zeros((p, kdim), jnp.float32)

    lane_blk = jax.lax.broadcasted_iota(jnp.int32, (p, kdim), 1) // d
    row_id = jax.lax.broadcasted_iota(jnp.int32, (p, kdim), 0)
    maskb = lane_blk == row_id

    b2row = jnp.concatenate(
        [jnp.broadcast_to(b20_ref[0], (batch, d)),
         jnp.broadcast_to(b20_ref[1], (batch, d))], axis=0)
    b2bd_s[...] = jnp.where(maskb, jnp.tile(b2row, (1, p)), 0.0)

    halves = ((slice(0, batch * d), slice(0, batch), mom_a, lr_a),
              (slice(batch * d, kdim), slice(batch, p), mom_b, lr_b))

    def step(it, carry):
        kt = k_scr[it]
        vt = v_scr[it]
        qt = q_scr[it]
        ktt = jnp.tile(kt, (1, p))
        vtt = jnp.tile(vt, (1, p))
        qtt = jnp.tile(qt, (1, p))
        kt_bd = jnp.where(maskb, ktt, 0.0)
        qt_bd = jnp.where(maskb, qtt, 0.0)
        w1v = w1v_s[...]
        w2v = w2v_s[...]
        b1 = b1_s[...]
        b2bd = b2bd_s[...]

        h1 = jnp.dot(kt_bd, w1v, preferred_element_type=jnp.float32) + b1
        cdf = 0.5 * (1.0 + jax.lax.erf(h1 * INVSQ2))
        a = h1 * cdf
        out_full = jax.lax.dot_general(
            a, w2v, (((1,), (1,)), ((), ())),
            preferred_element_type=jnp.float32)
        d_out_bd = jnp.where(maskb, out_full + b2bd - vtt, 0.0)
        d_a = jnp.dot(d_out_bd, w2v, preferred_element_type=jnp.float32)
        pdf = SQ2PI * jnp.exp(-0.5 * h1 * h1)
        d_h = d_a * (cdf + h1 * pdf)

        gw1v = jax.lax.dot_general(
            kt_bd, d_h, (((0,), (0,)), ((), ())),
            preferred_element_type=jnp.float32)
        gw2v = jax.lax.dot_general(
            d_out_bd, a, (((0,), (0,)), ((), ())),
            preferred_element_type=jnp.float32)

        w1n = []
        w2n = []
        b1n = []
        b2n = []
        for rs, bs, mo, l in halves:
            m1 = mo * mw1v_s[rs] + gw1v[rs]
            mw1v_s[rs] = m1
            w1h = w1v[rs] - l * m1
            w1v_s[rs] = w1h
            w1n.append(w1h)
            m2 = mo * mw2v_s[rs] + gw2v[rs]
            mw2v_s[rs] = m2
            w2h = w2v[rs] - l * m2
            w2v_s[rs] = w2h
            w2n.append(w2h)
            mb1 = mo * mb1_s[bs] + d_h[bs]
            mb1_s[bs] = mb1
            b1h = b1[bs] - l * mb1
            b1_s[bs] = b1h
            b1n.append(b1h)
            mb2 = mo * mb2bd_s[bs] + d_out_bd[bs]
            mb2bd_s[bs] = mb2
            b2h = b2bd[bs] - l * mb2
            b2bd_s[bs] = b2h
            b2n.append(b2h)
        w1v_n = jnp.concatenate(w1n, axis=0)
        w2v_n = jnp.concatenate(w2n, axis=0)
        b1_n = jnp.concatenate(b1n, axis=0)
        b2bd_n = jnp.concatenate(b2n, axis=0)

        h1q = jnp.dot(qt_bd, w1v_n, preferred_element_type=jnp.float32) + b1_n
        cdfq = 0.5 * (1.0 + jax.lax.erf(h1q * INVSQ2))
        aq = h1q * cdfq
        yq_full = jax.lax.dot_general(
            aq, w2v_n, (((1,), (1,)), ((), ())),
            preferred_element_type=jnp.float32)
        yq_bd = jnp.where(maskb, yq_full + b2bd_n, 0.0)
        yq = yq_bd[:, 0:d]
        for j in range(1, p):
            yq = yq + yq_bd[:, j * d:(j + 1) * d]
        y_ref[0, pl.ds(it, 1)] = yq[None]
        return carry

    jax.lax.fori_loop(0, seq, step, 0)


def _out_kernel(r_ref, x_ref, wo_ref, gw_ref, gb_ref, o_ref):
    rp = jax.lax.dot_general(r_ref[...], wo_ref[...], (((1,), (1,)), ((), ())),
                             precision=_HIGHEST,
                             preferred_element_type=jnp.float32)
    gt = jax.lax.dot_general(x_ref[...], gw_ref[...], (((1,), (1,)), ((), ())),
                             precision=_HIGHEST,
                             preferred_element_type=jnp.float32) + gb_ref[...]
    o_ref[...] = rp * jax.nn.sigmoid(gt)


def kernel(x, w_k, w_v, w_q, w_o, ln_k_g, ln_k_b, ln_v_g, ln_v_b, ln_q_g,
           ln_q_b, fc1_w, fc1_b, fc2_w, fc2_b, lr_scale, momentum_scale,
           gate_w, gate_b):
    batch, seq, dim = x.shape
    h = NUM_HEADS
    d = HEAD_DIM
    hid = HID
    p = 2 * batch

    x_tb = jnp.transpose(x, (1, 0, 2)).reshape(seq * batch, dim)
    fc1_wt = jnp.swapaxes(fc1_w, 1, 2)
    fc1_b3 = fc1_b.reshape(h, 1, hid)
    fc2_b3 = fc2_b.reshape(h, 1, d)
    lr2 = lr_scale.reshape(h, 1, 1)
    mom2 = momentum_scale.reshape(h, 1, 1)
    g2 = lambda v: v.reshape(1, d)

    scan = pl.pallas_call(
        functools.partial(_scan_kernel, batch=batch, seq=seq),
        grid=(2,),
        in_specs=[
            pl.BlockSpec((seq * batch, dim), lambda i: (0, 0)),
            pl.BlockSpec((2 * d, dim), lambda i: (i, 0)),
            pl.BlockSpec((2 * d, dim), lambda i: (i, 0)),
            pl.BlockSpec((2 * d, dim), lambda i: (i, 0)),
            pl.BlockSpec((1, d), lambda i: (0, 0)),
            pl.BlockSpec((1, d), lambda i: (0, 0)),
            pl.BlockSpec((1, d), lambda i: (0, 0)),
            pl.BlockSpec((1, d), lambda i: (0, 0)),
            pl.BlockSpec((1, d), lambda i: (0, 0)),
            pl.BlockSpec((1, d), lambda i: (0, 0)),
            pl.BlockSpec((2, d, hid), lambda i: (i, 0, 0)),
            pl.BlockSpec((2, 1, hid), lambda i: (i, 0, 0)),
            pl.BlockSpec((2, d, hid), lambda i: (i, 0, 0)),
            pl.BlockSpec((2, 1, d), lambda i: (i, 0, 0)),
            pl.BlockSpec((2, 1, 1), lambda i: (i, 0, 0)),
            pl.BlockSpec((2, 1, 1), lambda i: (i, 0, 0)),
        ],
        out_specs=pl.BlockSpec((1, seq, p, d), lambda i: (i, 0, 0, 0)),
        out_shape=jax.ShapeDtypeStruct((2, seq, p, d), jnp.float32),
        scratch_shapes=[
            pltpu.VMEM((seq, p, d), jnp.float32),
            pltpu.VMEM((seq, p, d), jnp.float32),
            pltpu.VMEM((seq, p, d), jnp.float32),
            pltpu.VMEM((p * d, hid), jnp.float32),
            pltpu.VMEM((p * d, hid), jnp.float32),
            pltpu.VMEM((p * d, hid), jnp.float32),
            pltpu.VMEM((p * d, hid), jnp.float32),
            pltpu.VMEM((p, hid), jnp.float32),
            pltpu.VMEM((p, p * d), jnp.float32),
            pltpu.VMEM((p, hid), jnp.float32),
            pltpu.VMEM((p, p * d), jnp.float32),
        ],
        compiler_params=pltpu.CompilerParams(
            dimension_semantics=("parallel",),
            vmem_limit_bytes=100 * 1024 * 1024,
        ),
    )
    ys = scan(x_tb, w_k, w_v, w_q, g2(ln_k_g), g2(ln_k_b), g2(ln_v_g),
              g2(ln_v_b), g2(ln_q_g), g2(ln_q_b), fc1_wt, fc1_b3, fc2_w,
              fc2_b3, lr2, mom2)

    r = jnp.transpose(ys.reshape(2, seq, 2, batch, d),
                      (3, 1, 0, 2, 4)).reshape(batch * seq, h * d)
    x2 = x.reshape(batch * seq, dim)

    blk = 256
    out = pl.pallas_call(
        _out_kernel,
        grid=(batch * seq // blk,),
        in_specs=[
            pl.BlockSpec((blk, h * d), lambda i: (i, 0)),
            pl.BlockSpec((blk, dim), lambda i: (i, 0)),
            pl.BlockSpec((dim, h * d), lambda i: (0, 0)),
            pl.BlockSpec((dim, dim), lambda i: (0, 0)),
            pl.BlockSpec((1, dim), lambda i: (0, 0)),
        ],
        out_specs=pl.BlockSpec((blk, dim), lambda i: (i, 0)),
        out_shape=jax.ShapeDtypeStruct((batch * seq, dim), jnp.float32),
        compiler_params=pltpu.CompilerParams(
            dimension_semantics=("parallel",),
        ),
    )(r, x2, w_o, gate_w, gate_b.reshape(1, dim))
    return out.reshape(batch, seq, dim)


# speedup vs baseline: 4.9422x; 1.4487x over previous
import functools
import math

import jax
import jax.numpy as jnp
from jax.experimental import pallas as pl
from jax.experimental.pallas import tpu as pltpu

DIM = 512
HEAD_DIM = 64
NUM_HEADS = 4
HID = 4 * HEAD_DIM
BASE_LR = 0.1
BASE_MOM = 0.9
LN_EPS = 1e-5
SQ2PI = math.sqrt(2.0 / math.pi)
INVSQ2 = 1.0 / math.sqrt(2.0)

_HIGHEST = jax.lax.Precision.HIGHEST


def _scan_kernel(x_ref, wk_ref, wv_ref, wq_ref, lnkg, lnkb, lnvg, lnvb,
                 lnqg, lnqb, w1t0_ref, b10_ref, w20_ref, b20_ref, lr_ref,
                 mom_ref, y_ref, k_scr, v_scr, q_scr, w1v_s, w2v_s, mw1v_s,
                 mw2v_s, b1_s, b2bd_s, mb1_s, mb2bd_s, *, batch, seq):
    d = HEAD_DIM
    hid = HID
    p = 2 * batch
    kdim = p * d

    xb = x_ref[...]

    def proj(w_ref, g_ref, bb_ref, scr):
        for h2 in range(2):
            w = w_ref[h2 * d:(h2 + 1) * d, :]
            t = jax.lax.dot_general(xb, w, (((1,), (1,)), ((), ())),
                                    precision=_HIGHEST,
                                    preferred_element_type=jnp.float32)
            mu = jnp.mean(t, axis=1, keepdims=True)
            var = jnp.mean((t - mu) ** 2, axis=1, keepdims=True)
            tn = (t - mu) / jnp.sqrt(var + LN_EPS) * g_ref[...] + bb_ref[...]
            scr[:, h2 * batch:(h2 + 1) * batch, :] = tn.reshape(seq, batch, d)

    proj(wk_ref, lnkg, lnkb, k_scr)
    proj(wv_ref, lnvg, lnvb, v_scr)
    proj(wq_ref, lnqg, lnqb, q_scr)

    lr_a = BASE_LR * jax.nn.sigmoid(lr_ref[0, 0, 0])
    lr_b = BASE_LR * jax.nn.sigmoid(lr_ref[1, 0, 0])
    mom_a = BASE_MOM * jax.nn.sigmoid(mom_ref[0, 0, 0])
    mom_b = BASE_MOM * jax.nn.sigmoid(mom_ref[1, 0, 0])

    for i in range(batch):
        w1v_s[i * d:(i + 1) * d] = w1t0_ref[0]
        w1v_s[(batch + i) * d:(batch + i + 1) * d] = w1t0_ref[1]
        w2v_s[i * d:(i + 1) * d] = w20_ref[0]
        w2v_s[(batch + i) * d:(batch + i + 1) * d] = w20_ref[1]
    mw1v_s[...] = jnp.zeros((kdim, hid), jnp.float32)
    mw2v_s[...] = jnp.zeros((kdim, hid), jnp.float32)
    b1_s[0:batch] = jnp.broadcast_to(b10_ref[0], (batch, hid))
    b1_s[batch:p] = jnp.broadcast_to(b10_ref[1], (batch, hid))
    mb1_s[...] = jnp.zeros((p, hid), jnp.float32)
    mb2bd_s[...] = jnp.zeros((p, kdim), jnp.float32)

    lane_blk = jax.lax.broadcasted_iota(jnp.int32, (p, kdim), 1) // d
    row_id = jax.lax.broadcasted_iota(jnp.int32, (p, kdim), 0)
    maskb = lane_blk == row_id
    lane_blk2 = jax.lax.broadcasted_iota(jnp.int32, (2 * p, kdim), 1) // d
    row_id2 = jax.lax.broadcasted_iota(jnp.int32, (2 * p, kdim), 0) % p
    maskb2 = lane_blk2 == row_id2

    b2row = jnp.concatenate(
        [jnp.broadcast_to(b20_ref[0], (batch, d)),
         jnp.broadcast_to(b20_ref[1], (batch, d))], axis=0)
    b2bd_s[...] = jnp.where(maskb, jnp.tile(b2row, (1, p)), 0.0)

    halves = ((slice(0, batch * d), slice(0, batch), mom_a, lr_a),
              (slice(batch * d, kdim), slice(batch, p), mom_b, lr_b))

    def seg_sum(bd):
        yq = bd[:, 0:d]
        for j in range(1, p):
            yq = yq + bd[:, j * d:(j + 1) * d]
        return yq

    def step(it, carry, with_retrieve):
        kt = k_scr[it]
        vt = v_scr[it]
        if with_retrieve:
            qp = q_scr[it - 1]
            inp = jnp.concatenate([qp, kt], axis=0)
            mask_n = maskb2
        else:
            inp = kt
            mask_n = maskb
        inp_bd = jnp.where(mask_n, jnp.tile(inp, (1, p)), 0.0)
        w1v = w1v_s[...]
        w2v = w2v_s[...]
        b1 = b1_s[...]
        b2bd = b2bd_s[...]
        if with_retrieve:
            b1_n = jnp.concatenate([b1, b1], axis=0)
            b2bd_n = jnp.concatenate([b2bd, b2bd], axis=0)
        else:
            b1_n = b1
            b2bd_n = b2bd

        hh = jnp.dot(inp_bd, w1v, preferred_element_type=jnp.float32) + b1_n
        cdf = 0.5 * (1.0 + jax.lax.erf(hh * INVSQ2))
        aa = hh * cdf
        of = jax.lax.dot_general(
            aa, w2v, (((1,), (1,)), ((), ())),
            preferred_element_type=jnp.float32) + b2bd_n

        if with_retrieve:
            yq_bd = jnp.where(maskb, of[0:p], 0.0)
            y_ref[0, pl.ds(it - 1, 1)] = seg_sum(yq_bd)[None]
            out_rows = of[p:2 * p]
            h1 = hh[p:2 * p]
            cdf_f = cdf[p:2 * p]
            a = aa[p:2 * p]
            kt_bd = inp_bd[p:2 * p]
        else:
            out_rows = of
            h1 = hh
            cdf_f = cdf
            a = aa
            kt_bd = inp_bd

        vtt = jnp.tile(vt, (1, p))
        d_out_bd = jnp.where(maskb, out_rows - vtt, 0.0)
        d_a = jnp.dot(d_out_bd, w2v, preferred_element_type=jnp.float32)
        pdf = SQ2PI * jnp.exp(-0.5 * h1 * h1)
        d_h = d_a * (cdf_f + h1 * pdf)

        gw1v = jax.lax.dot_general(
            kt_bd, d_h, (((0,), (0,)), ((), ())),
            preferred_element_type=jnp.float32)
        gw2v = jax.lax.dot_general(
            d_out_bd, a, (((0,), (0,)), ((), ())),
            preferred_element_type=jnp.float32)

        for rs, bs, mo, l in halves:
            m1 = mo * mw1v_s[rs] + gw1v[rs]
            mw1v_s[rs] = m1
            w1v_s[rs] = w1v[rs] - l * m1
            m2 = mo * mw2v_s[rs] + gw2v[rs]
            mw2v_s[rs] = m2
            w2v_s[rs] = w2v[rs] - l * m2
            mb1 = mo * mb1_s[bs] + d_h[bs]
            mb1_s[bs] = mb1
            b1_s[bs] = b1[bs] - l * mb1
            mb2 = mo * mb2bd_s[bs] + d_out_bd[bs]
            mb2bd_s[bs] = mb2
            b2bd_s[bs] = b2bd[bs] - l * mb2
        return carry

    step(0, 0, with_retrieve=False)
    jax.lax.fori_loop(1, seq,
                      functools.partial(step, with_retrieve=True), 0)

    qp = q_scr[seq - 1]
    qp_bd = jnp.where(maskb, jnp.tile(qp, (1, p)), 0.0)
    h1q = jnp.dot(qp_bd, w1v_s[...], preferred_element_type=jnp.float32) \
        + b1_s[...]
    aq = h1q * (0.5 * (1.0 + jax.lax.erf(h1q * INVSQ2)))
    yq_full = jax.lax.dot_general(
        aq, w2v_s[...], (((1,), (1,)), ((), ())),
        preferred_element_type=jnp.float32) + b2bd_s[...]
    yq_bd = jnp.where(maskb, yq_full, 0.0)
    y_ref[0, pl.ds(seq - 1, 1)] = seg_sum(yq_bd)[None]


def _out_kernel(r_ref, x_ref, wo_ref, gw_ref, gb_ref, o_ref):
    rp = jax.lax.dot_general(r_ref[...], wo_ref[...], (((1,), (1,)), ((), ())),
                             precision=_HIGHEST,
                             preferred_element_type=jnp.float32)
    gt = jax.lax.dot_general(x_ref[...], gw_ref[...], (((1,), (1,)), ((), ())),
                             precision=_HIGHEST,
                             preferred_element_type=jnp.float32) + gb_ref[...]
    o_ref[...] = rp * jax.nn.sigmoid(gt)


def kernel(x, w_k, w_v, w_q, w_o, ln_k_g, ln_k_b, ln_v_g, ln_v_b, ln_q_g,
           ln_q_b, fc1_w, fc1_b, fc2_w, fc2_b, lr_scale, momentum_scale,
           gate_w, gate_b):
    batch, seq, dim = x.shape
    h = NUM_HEADS
    d = HEAD_DIM
    hid = HID
    p = 2 * batch

    x_tb = jnp.transpose(x, (1, 0, 2)).reshape(seq * batch, dim)
    fc1_wt = jnp.swapaxes(fc1_w, 1, 2)
    fc1_b3 = fc1_b.reshape(h, 1, hid)
    fc2_b3 = fc2_b.reshape(h, 1, d)
    lr2 = lr_scale.reshape(h, 1, 1)
    mom2 = momentum_scale.reshape(h, 1, 1)
    g2 = lambda v: v.reshape(1, d)

    scan = pl.pallas_call(
        functools.partial(_scan_kernel, batch=batch, seq=seq),
        grid=(2,),
        in_specs=[
            pl.BlockSpec((seq * batch, dim), lambda i: (0, 0)),
            pl.BlockSpec((2 * d, dim), lambda i: (i, 0)),
            pl.BlockSpec((2 * d, dim), lambda i: (i, 0)),
            pl.BlockSpec((2 * d, dim), lambda i: (i, 0)),
            pl.BlockSpec((1, d), lambda i: (0, 0)),
            pl.BlockSpec((1, d), lambda i: (0, 0)),
            pl.BlockSpec((1, d), lambda i: (0, 0)),
            pl.BlockSpec((1, d), lambda i: (0, 0)),
            pl.BlockSpec((1, d), lambda i: (0, 0)),
            pl.BlockSpec((1, d), lambda i: (0, 0)),
            pl.BlockSpec((2, d, hid), lambda i: (i, 0, 0)),
            pl.BlockSpec((2, 1, hid), lambda i: (i, 0, 0)),
            pl.BlockSpec((2, d, hid), lambda i: (i, 0, 0)),
            pl.BlockSpec((2, 1, d), lambda i: (i, 0, 0)),
            pl.BlockSpec((2, 1, 1), lambda i: (i, 0, 0)),
            pl.BlockSpec((2, 1, 1), lambda i: (i, 0, 0)),
        ],
        out_specs=pl.BlockSpec((1, seq, p, d), lambda i: (i, 0, 0, 0)),
        out_shape=jax.ShapeDtypeStruct((2, seq, p, d), jnp.float32),
        scratch_shapes=[
            pltpu.VMEM((seq, p, d), jnp.float32),
            pltpu.VMEM((seq, p, d), jnp.float32),
            pltpu.VMEM((seq, p, d), jnp.float32),
            pltpu.VMEM((p * d, hid), jnp.float32),
            pltpu.VMEM((p * d, hid), jnp.float32),
            pltpu.VMEM((p * d, hid), jnp.float32),
            pltpu.VMEM((p * d, hid), jnp.float32),
            pltpu.VMEM((p, hid), jnp.float32),
            pltpu.VMEM((p, p * d), jnp.float32),
            pltpu.VMEM((p, hid), jnp.float32),
            pltpu.VMEM((p, p * d), jnp.float32),
        ],
        compiler_params=pltpu.CompilerParams(
            dimension_semantics=("parallel",),
            vmem_limit_bytes=100 * 1024 * 1024,
        ),
    )
    ys = scan(x_tb, w_k, w_v, w_q, g2(ln_k_g), g2(ln_k_b), g2(ln_v_g),
              g2(ln_v_b), g2(ln_q_g), g2(ln_q_b), fc1_wt, fc1_b3, fc2_w,
              fc2_b3, lr2, mom2)

    r = jnp.transpose(ys.reshape(2, seq, 2, batch, d),
                      (3, 1, 0, 2, 4)).reshape(batch * seq, h * d)
    x2 = x.reshape(batch * seq, dim)

    blk = 256
    out = pl.pallas_call(
        _out_kernel,
        grid=(batch * seq // blk,),
        in_specs=[
            pl.BlockSpec((blk, h * d), lambda i: (i, 0)),
            pl.BlockSpec((blk, dim), lambda i: (i, 0)),
            pl.BlockSpec((dim, h * d), lambda i: (0, 0)),
            pl.BlockSpec((dim, dim), lambda i: (0, 0)),
            pl.BlockSpec((1, dim), lambda i: (0, 0)),
        ],
        out_specs=pl.BlockSpec((blk, dim), lambda i: (i, 0)),
        out_shape=jax.ShapeDtypeStruct((batch * seq, dim), jnp.float32),
        compiler_params=pltpu.CompilerParams(
            dimension_semantics=("parallel",),
        ),
    )(r, x2, w_o, gate_w, gate_b.reshape(1, dim))
    return out.reshape(batch, seq, dim)


# speedup vs baseline: 5.0993x; 1.0318x over previous
import functools
import math

import jax
import jax.numpy as jnp
from jax.experimental import pallas as pl
from jax.experimental.pallas import tpu as pltpu

DIM = 512
HEAD_DIM = 64
NUM_HEADS = 4
HID = 4 * HEAD_DIM
BASE_LR = 0.1
BASE_MOM = 0.9
LN_EPS = 1e-5
SQ2PI = math.sqrt(2.0 / math.pi)
INVSQ2 = 1.0 / math.sqrt(2.0)

_HIGHEST = jax.lax.Precision.HIGHEST


def _scan_kernel(x_ref, wk_ref, wv_ref, wq_ref, lnkg, lnkb, lnvg, lnvb,
                 lnqg, lnqb, w1t0_ref, b10_ref, w20_ref, b20_ref, lr_ref,
                 mom_ref, y_ref, k_scr, v_scr, q_scr, w1v_s, w2v_s, mw1v_s,
                 mw2v_s, b1_s, b2bd_s, mb1_s, mb2bd_s, *, batch, seq):
    d = HEAD_DIM
    hid = HID
    p = 2 * batch
    kdim = p * d

    xb = x_ref[...]

    def proj(w_ref, g_ref, bb_ref, scr):
        for h2 in range(2):
            w = w_ref[h2 * d:(h2 + 1) * d, :]
            t = jax.lax.dot_general(xb, w, (((1,), (1,)), ((), ())),
                                    precision=_HIGHEST,
                                    preferred_element_type=jnp.float32)
            mu = jnp.mean(t, axis=1, keepdims=True)
            var = jnp.mean((t - mu) ** 2, axis=1, keepdims=True)
            tn = (t - mu) / jnp.sqrt(var + LN_EPS) * g_ref[...] + bb_ref[...]
            scr[:, h2 * batch:(h2 + 1) * batch, :] = tn.reshape(seq, batch, d)

    proj(wk_ref, lnkg, lnkb, k_scr)
    proj(wv_ref, lnvg, lnvb, v_scr)
    proj(wq_ref, lnqg, lnqb, q_scr)

    lr_a = BASE_LR * jax.nn.sigmoid(lr_ref[0, 0, 0])
    lr_b = BASE_LR * jax.nn.sigmoid(lr_ref[1, 0, 0])
    mom_a = BASE_MOM * jax.nn.sigmoid(mom_ref[0, 0, 0])
    mom_b = BASE_MOM * jax.nn.sigmoid(mom_ref[1, 0, 0])

    for i in range(batch):
        w1v_s[i * d:(i + 1) * d] = w1t0_ref[0]
        w1v_s[(batch + i) * d:(batch + i + 1) * d] = w1t0_ref[1]
        w2v_s[i * d:(i + 1) * d] = w20_ref[0]
        w2v_s[(batch + i) * d:(batch + i + 1) * d] = w20_ref[1]
    mw1v_s[...] = jnp.zeros((kdim, hid), jnp.float32)
    mw2v_s[...] = jnp.zeros((kdim, hid), jnp.float32)
    b1_s[0:batch] = jnp.broadcast_to(b10_ref[0], (batch, hid))
    b1_s[batch:p] = jnp.broadcast_to(b10_ref[1], (batch, hid))
    mb1_s[...] = jnp.zeros((p, hid), jnp.float32)
    mb2bd_s[...] = jnp.zeros((p, kdim), jnp.float32)

    lane_blk = jax.lax.broadcasted_iota(jnp.int32, (p, kdim), 1) // d
    row_id = jax.lax.broadcasted_iota(jnp.int32, (p, kdim), 0)
    maskb = lane_blk == row_id
    lane_blk2 = jax.lax.broadcasted_iota(jnp.int32, (2 * p, kdim), 1) // d
    row_id2 = jax.lax.broadcasted_iota(jnp.int32, (2 * p, kdim), 0) % p
    maskb2 = lane_blk2 == row_id2

    b2row = jnp.concatenate(
        [jnp.broadcast_to(b20_ref[0], (batch, d)),
         jnp.broadcast_to(b20_ref[1], (batch, d))], axis=0)
    b2bd_s[...] = jnp.where(maskb, jnp.tile(b2row, (1, p)), 0.0)

    halves = ((slice(0, batch * d), slice(0, batch), mom_a, lr_a),
              (slice(batch * d, kdim), slice(batch, p), mom_b, lr_b))

    def seg_sum(bd):
        yq = bd[:, 0:d]
        for j in range(1, p):
            yq = yq + bd[:, j * d:(j + 1) * d]
        return yq

    def step(it, carry, with_retrieve):
        kt = k_scr[it]
        vt = v_scr[it]
        if with_retrieve:
            qp = q_scr[it - 1]
            inp = jnp.concatenate([qp, kt], axis=0)
            mask_n = maskb2
        else:
            inp = kt
            mask_n = maskb
        inp_bd = jnp.where(mask_n, jnp.tile(inp, (1, p)), 0.0)
        w1v = w1v_s[...]
        w2v = w2v_s[...]
        b1 = b1_s[...]
        b2bd = b2bd_s[...]
        if with_retrieve:
            b1_n = jnp.concatenate([b1, b1], axis=0)
            b2bd_n = jnp.concatenate([b2bd, b2bd], axis=0)
        else:
            b1_n = b1
            b2bd_n = b2bd

        hh = jnp.dot(inp_bd, w1v, preferred_element_type=jnp.float32) + b1_n
        cdf = 0.5 * (1.0 + jax.lax.erf(hh * INVSQ2))
        aa = hh * cdf
        of = jax.lax.dot_general(
            aa, w2v, (((1,), (1,)), ((), ())),
            preferred_element_type=jnp.float32) + b2bd_n

        if with_retrieve:
            yq_bd = jnp.where(maskb, of[0:p], 0.0)
            y_ref[0, pl.ds(it - 1, 1)] = seg_sum(yq_bd)[None]
            out_rows = of[p:2 * p]
            h1 = hh[p:2 * p]
            cdf_f = cdf[p:2 * p]
            a = aa[p:2 * p]
            kt_bd = inp_bd[p:2 * p]
        else:
            out_rows = of
            h1 = hh
            cdf_f = cdf
            a = aa
            kt_bd = inp_bd

        vtt = jnp.tile(vt, (1, p))
        d_out_bd = jnp.where(maskb, out_rows - vtt, 0.0)
        d_a = jnp.dot(d_out_bd, w2v, preferred_element_type=jnp.float32)
        pdf = SQ2PI * jnp.exp(-0.5 * h1 * h1)
        d_h = d_a * (cdf_f + h1 * pdf)

        gw1v = jax.lax.dot_general(
            kt_bd, d_h, (((0,), (0,)), ((), ())),
            preferred_element_type=jnp.float32)
        gw2v = jax.lax.dot_general(
            d_out_bd, a, (((0,), (0,)), ((), ())),
            preferred_element_type=jnp.float32)

        for rs, bs, mo, l in halves:
            m1 = mo * mw1v_s[rs] + gw1v[rs]
            mw1v_s[rs] = m1
            w1v_s[rs] = w1v[rs] - l * m1
            m2 = mo * mw2v_s[rs] + gw2v[rs]
            mw2v_s[rs] = m2
            w2v_s[rs] = w2v[rs] - l * m2
            mb1 = mo * mb1_s[bs] + d_h[bs]
            mb1_s[bs] = mb1
            b1_s[bs] = b1[bs] - l * mb1
            mb2 = mo * mb2bd_s[bs] + d_out_bd[bs]
            mb2bd_s[bs] = mb2
            b2bd_s[bs] = b2bd[bs] - l * mb2
        return carry

    step(0, 0, with_retrieve=False)
    jax.lax.fori_loop(1, seq,
                      functools.partial(step, with_retrieve=True), 0,
                      unroll=2)

    qp = q_scr[seq - 1]
    qp_bd = jnp.where(maskb, jnp.tile(qp, (1, p)), 0.0)
    h1q = jnp.dot(qp_bd, w1v_s[...], preferred_element_type=jnp.float32) \
        + b1_s[...]
    aq = h1q * (0.5 * (1.0 + jax.lax.erf(h1q * INVSQ2)))
    yq_full = jax.lax.dot_general(
        aq, w2v_s[...], (((1,), (1,)), ((), ())),
        preferred_element_type=jnp.float32) + b2bd_s[...]
    yq_bd = jnp.where(maskb, yq_full, 0.0)
    y_ref[0, pl.ds(seq - 1, 1)] = seg_sum(yq_bd)[None]


def _out_kernel(r_ref, x_ref, wo_ref, gw_ref, gb_ref, o_ref):
    rp = jax.lax.dot_general(r_ref[...], wo_ref[...], (((1,), (1,)), ((), ())),
                             precision=_HIGHEST,
                             preferred_element_type=jnp.float32)
    gt = jax.lax.dot_general(x_ref[...], gw_ref[...], (((1,), (1,)), ((), ())),
                             precision=_HIGHEST,
                             preferred_element_type=jnp.float32) + gb_ref[...]
    o_ref[...] = rp * jax.nn.sigmoid(gt)


def kernel(x, w_k, w_v, w_q, w_o, ln_k_g, ln_k_b, ln_v_g, ln_v_b, ln_q_g,
           ln_q_b, fc1_w, fc1_b, fc2_w, fc2_b, lr_scale, momentum_scale,
           gate_w, gate_b):
    batch, seq, dim = x.shape
    h = NUM_HEADS
    d = HEAD_DIM
    hid = HID
    p = 2 * batch

    x_tb = jnp.transpose(x, (1, 0, 2)).reshape(seq * batch, dim)
    fc1_wt = jnp.swapaxes(fc1_w, 1, 2)
    fc1_b3 = fc1_b.reshape(h, 1, hid)
    fc2_b3 = fc2_b.reshape(h, 1, d)
    lr2 = lr_scale.reshape(h, 1, 1)
    mom2 = momentum_scale.reshape(h, 1, 1)
    g2 = lambda v: v.reshape(1, d)

    scan = pl.pallas_call(
        functools.partial(_scan_kernel, batch=batch, seq=seq),
        grid=(2,),
        in_specs=[
            pl.BlockSpec((seq * batch, dim), lambda i: (0, 0)),
            pl.BlockSpec((2 * d, dim), lambda i: (i, 0)),
            pl.BlockSpec((2 * d, dim), lambda i: (i, 0)),
            pl.BlockSpec((2 * d, dim), lambda i: (i, 0)),
            pl.BlockSpec((1, d), lambda i: (0, 0)),
            pl.BlockSpec((1, d), lambda i: (0, 0)),
            pl.BlockSpec((1, d), lambda i: (0, 0)),
            pl.BlockSpec((1, d), lambda i: (0, 0)),
            pl.BlockSpec((1, d), lambda i: (0, 0)),
            pl.BlockSpec((1, d), lambda i: (0, 0)),
            pl.BlockSpec((2, d, hid), lambda i: (i, 0, 0)),
            pl.BlockSpec((2, 1, hid), lambda i: (i, 0, 0)),
            pl.BlockSpec((2, d, hid), lambda i: (i, 0, 0)),
            pl.BlockSpec((2, 1, d), lambda i: (i, 0, 0)),
            pl.BlockSpec((2, 1, 1), lambda i: (i, 0, 0)),
            pl.BlockSpec((2, 1, 1), lambda i: (i, 0, 0)),
        ],
        out_specs=pl.BlockSpec((1, seq, p, d), lambda i: (i, 0, 0, 0)),
        out_shape=jax.ShapeDtypeStruct((2, seq, p, d), jnp.float32),
        scratch_shapes=[
            pltpu.VMEM((seq, p, d), jnp.float32),
            pltpu.VMEM((seq, p, d), jnp.float32),
            pltpu.VMEM((seq, p, d), jnp.float32),
            pltpu.VMEM((p * d, hid), jnp.float32),
            pltpu.VMEM((p * d, hid), jnp.float32),
            pltpu.VMEM((p * d, hid), jnp.float32),
            pltpu.VMEM((p * d, hid), jnp.float32),
            pltpu.VMEM((p, hid), jnp.float32),
            pltpu.VMEM((p, p * d), jnp.float32),
            pltpu.VMEM((p, hid), jnp.float32),
            pltpu.VMEM((p, p * d), jnp.float32),
        ],
        compiler_params=pltpu.CompilerParams(
            dimension_semantics=("parallel",),
            vmem_limit_bytes=100 * 1024 * 1024,
            flags={"XLA_TPU_STORE_TO_LOAD_FORWARDING_WINDOW": 12288},
        ),
    )
    ys = scan(x_tb, w_k, w_v, w_q, g2(ln_k_g), g2(ln_k_b), g2(ln_v_g),
              g2(ln_v_b), g2(ln_q_g), g2(ln_q_b), fc1_wt, fc1_b3, fc2_w,
              fc2_b3, lr2, mom2)

    r = jnp.transpose(ys.reshape(2, seq, 2, batch, d),
                      (3, 1, 0, 2, 4)).reshape(batch * seq, h * d)
    x2 = x.reshape(batch * seq, dim)

    blk = 256
    out = pl.pallas_call(
        _out_kernel,
        grid=(batch * seq // blk,),
        in_specs=[
            pl.BlockSpec((blk, h * d), lambda i: (i, 0)),
            pl.BlockSpec((blk, dim), lambda i: (i, 0)),
            pl.BlockSpec((dim, h * d), lambda i: (0, 0)),
            pl.BlockSpec((dim, dim), lambda i: (0, 0)),
            pl.BlockSpec((1, dim), lambda i: (0, 0)),
        ],
        out_specs=pl.BlockSpec((blk, dim), lambda i: (i, 0)),
        out_shape=jax.ShapeDtypeStruct((batch * seq, dim), jnp.float32),
        compiler_params=pltpu.CompilerParams(
            dimension_semantics=("parallel",),
        ),
    )(r, x2, w_o, gate_w, gate_b.reshape(1, dim))
    return out.reshape(batch, seq, dim)
